# Initial kernel scaffold; baseline (speedup 1.0000x reference)
import jax
import jax.numpy as jnp
from jax.experimental import pallas as pl


def kernel(g2m_efeat, grid_nfeat, mesh_nfeat, eW0, eb0, eW1, eb1, eg, ebt, sW0, sb0, sW1, sb1, sg, sbt, dW0, db0, dW1, db1, dg, dbt, src, dst):
    raise NotImplementedError("write your pallas kernel here")


# speedup vs baseline: 3.0524x; 3.0524x over previous
import functools

import jax
import jax.numpy as jnp
from jax import lax
from jax.experimental import pallas as pl
from jax.experimental.pallas import tpu as pltpu
from jax.experimental.pallas import tpu_sc as plsc

N_GRID = 10000
N_MESH = 10000
E = 320000
D = 128
H = 128

NC = 2
NS = 16
NW = NC * NS
EPW = E // NW
CHUNK = 80
NCHUNK = EPW // CHUNK


def _ln(y, g, b):
    m = jnp.mean(y, axis=-1, keepdims=True)
    v = jnp.mean((y - m) ** 2, axis=-1, keepdims=True)
    return (y - m) * lax.rsqrt(v + 1e-5) * g + b


def _silu(x):
    return x * jax.nn.sigmoid(x)


def _s1_body(grid_ref, mesh_ref, w0b_ref, w0c_ref, eb0_ref,
             sw0_ref, sb0_ref, sw1_ref, sb1_ref, sg_ref, sbt_ref,
             pg_ref, pm_ref, gout_ref):
    g = grid_ref[...]
    m = mesh_ref[...]
    pg_ref[...] = g @ w0b_ref[...] + eb0_ref[...]
    pm_ref[...] = m @ w0c_ref[...]
    h = _silu(g @ sw0_ref[...] + sb0_ref[...])
    y = h @ sw1_ref[...] + sb1_ref[...]
    gout_ref[...] = g + _ln(y, sg_ref[...], sbt_ref[...])


def _stage1(grid_nfeat, mesh_nfeat, w0b, w0c, eb0, sw0, sb0, sw1, sb1, sg, sbt):
    R = 1000
    row = pl.BlockSpec((R, D), lambda i: (i, 0))
    mat = pl.BlockSpec((D, H), lambda i: (0, 0))
    vec = pl.BlockSpec((1, H), lambda i: (0, 0))
    f32 = jnp.float32
    return pl.pallas_call(
        _s1_body,
        grid=(N_GRID // R,),
        in_specs=[row, row, mat, mat, vec, mat, vec, mat, vec, vec, vec],
        out_specs=[row, row, row],
        out_shape=[jax.ShapeDtypeStruct((N_GRID, H), f32),
                   jax.ShapeDtypeStruct((N_MESH, H), f32),
                   jax.ShapeDtypeStruct((N_GRID, D), f32)],
    )(grid_nfeat, mesh_nfeat, w0b, w0c, eb0, sw0, sb0, sw1, sb1, sg, sbt)


def _gather_body(pg_hbm, pm_hbm, src_hbm, dst_hbm, out_hbm,
                 idxs_v, idxd_v, bufa, bufb, sema, semb):
    c = lax.axis_index("c")
    s = lax.axis_index("s")
    w = s * NC + c
    pltpu.sync_copy(src_hbm.at[w], idxs_v)
    pltpu.sync_copy(dst_hbm.at[w], idxd_v)

    def body(j, carry):
        ca = pltpu.async_copy(pg_hbm.at[idxs_v.at[j]], bufa, sema)
        cb = pltpu.async_copy(pm_hbm.at[idxd_v.at[j]], bufb, semb)
        ca.wait()
        cb.wait()

        def add_row(r, carry2):
            for q in range(D // 16):
                sl = pl.ds(q * 16, 16)
                bufa[r, sl] = bufa[r, sl] + bufb[r, sl]
            return carry2

        lax.fori_loop(0, CHUNK, add_row, 0, unroll=False)
        pltpu.sync_copy(bufa, out_hbm.at[pl.ds(w * EPW + j * CHUNK, CHUNK)])
        return carry

    lax.fori_loop(0, NCHUNK, body, 0, unroll=False)


def _stage2(pg, pm, src_r, dst_r):
    mesh = plsc.VectorSubcoreMesh(core_axis_name="c", subcore_axis_name="s")
    fn = pl.kernel(
        _gather_body,
        out_type=jax.ShapeDtypeStruct((E, D), jnp.float32),
        mesh=mesh,
        scratch_types=[
            pltpu.VMEM((NCHUNK, CHUNK), jnp.int32),
            pltpu.VMEM((NCHUNK, CHUNK), jnp.int32),
            pltpu.VMEM((CHUNK, D), jnp.float32),
            pltpu.VMEM((CHUNK, D), jnp.float32),
            pltpu.SemaphoreType.DMA,
            pltpu.SemaphoreType.DMA,
        ],
    )
    return fn(pg, pm, src_r, dst_r)


def _edge_body(ef_ref, gs_ref, w0a_ref, w1_ref, eb1_ref, eg_ref, ebt_ref,
               out_ref):
    h = _silu(ef_ref[...] @ w0a_ref[...] + gs_ref[...])
    y = h @ w1_ref[...] + eb1_ref[...]
    out_ref[...] = _ln(y, eg_ref[...], ebt_ref[...])


def _stage3(efeat, gsum, w0a, w1, eb1, eg, ebt):
    R = 1280
    row = pl.BlockSpec((R, D), lambda i: (i, 0))
    mat = pl.BlockSpec((D, H), lambda i: (0, 0))
    vec = pl.BlockSpec((1, H), lambda i: (0, 0))
    return pl.pallas_call(
        _edge_body,
        grid=(E // R,),
        in_specs=[row, row, mat, mat, vec, vec, vec],
        out_specs=row,
        out_shape=jax.ShapeDtypeStruct((E, D), jnp.float32),
    )(efeat, gsum, w0a, w1, eb1, eg, ebt)


def _scatter_body(y_hbm, dst_hbm, zeros_hbm, out_hbm, idx_v, upd_v, acc_sh):
    c = lax.axis_index("c")
    s = lax.axis_index("s")
    w = s * NC + c

    @pl.when(s == 0)
    def _():
        pltpu.sync_copy(zeros_hbm, acc_sh)

    plsc.subcore_barrier()
    pltpu.sync_copy(dst_hbm.at[w], idx_v)

    def body(j, carry):
        pltpu.sync_copy(y_hbm.at[pl.ds(w * EPW + j * CHUNK, CHUNK)], upd_v)
        pltpu.sync_copy(upd_v, acc_sh.at[idx_v.at[j]], add=True)
        return carry

    lax.fori_loop(0, NCHUNK, body, 0, unroll=False)
    plsc.subcore_barrier()

    @pl.when(s < 15)
    def _():
        pltpu.sync_copy(acc_sh.at[pl.ds(s * 640, 640)],
                        out_hbm.at[c, pl.ds(s * 640, 640)])

    @pl.when(s == 15)
    def _():
        pltpu.sync_copy(acc_sh.at[pl.ds(9600, 400)],
                        out_hbm.at[c, pl.ds(9600, 400)])


def _stage4(mlp_e, dst_r, zeros):
    mesh = plsc.VectorSubcoreMesh(core_axis_name="c", subcore_axis_name="s")
    fn = pl.kernel(
        _scatter_body,
        out_type=jax.ShapeDtypeStruct((NC, N_MESH, D), jnp.float32),
        mesh=mesh,
        scratch_types=[
            pltpu.VMEM((NCHUNK, CHUNK), jnp.int32),
            pltpu.VMEM((CHUNK, D), jnp.float32),
            pltpu.VMEM_SHARED((N_MESH, D), jnp.float32),
        ],
    )
    return fn(mlp_e, dst_r, zeros)


def _s5_body(parts_ref, mesh_ref, dw0a_ref, dw0b_ref, db0_ref,
             dw1_ref, db1_ref, dg_ref, dbt_ref, out_ref):
    agg = parts_ref[0] + parts_ref[1]
    m = mesh_ref[...]
    h = _silu(agg @ dw0a_ref[...] + m @ dw0b_ref[...] + db0_ref[...])
    y = h @ dw1_ref[...] + db1_ref[...]
    out_ref[...] = m + _ln(y, dg_ref[...], dbt_ref[...])


def _stage5(parts, mesh_nfeat, dw0a, dw0b, db0, dw1, db1, dg, dbt):
    R = 1000
    row = pl.BlockSpec((R, D), lambda i: (i, 0))
    mat = pl.BlockSpec((D, H), lambda i: (0, 0))
    vec = pl.BlockSpec((1, H), lambda i: (0, 0))
    pspec = pl.BlockSpec((NC, R, D), lambda i: (0, i, 0))
    return pl.pallas_call(
        _s5_body,
        grid=(N_MESH // R,),
        in_specs=[pspec, row, mat, mat, vec, mat, vec, vec, vec],
        out_specs=row,
        out_shape=jax.ShapeDtypeStruct((N_MESH, D), jnp.float32),
    )(parts, mesh_nfeat, dw0a, dw0b, db0, dw1, db1, dg, dbt)


def kernel(g2m_efeat, grid_nfeat, mesh_nfeat,
           eW0, eb0, eW1, eb1, eg, ebt,
           sW0, sb0, sW1, sb1, sg, sbt,
           dW0, db0, dW1, db1, dg, dbt,
           src, dst):
    w0a, w0b, w0c = eW0[:D], eW0[D:2 * D], eW0[2 * D:]
    dw0a, dw0b = dW0[:D], dW0[D:]
    r2 = lambda v: v.reshape(1, -1)

    pg, pm, grid_out = _stage1(grid_nfeat, mesh_nfeat, w0b, w0c, r2(eb0),
                               sW0, r2(sb0), sW1, r2(sb1), r2(sg), r2(sbt))

    src_r = src.reshape(NW, NCHUNK, CHUNK)
    dst_r = dst.reshape(NW, NCHUNK, CHUNK)
    gsum = _stage2(pg, pm, src_r, dst_r)

    mlp_e = _stage3(g2m_efeat, gsum, w0a, eW1, r2(eb1), r2(eg), r2(ebt))

    zeros = jnp.zeros((N_MESH, D), jnp.float32)
    parts = _stage4(mlp_e, dst_r, zeros)

    mesh_out = _stage5(parts, mesh_nfeat, dw0a, dw0b, r2(db0),
                       dW1, r2(db1), r2(dg), r2(dbt))
    return (grid_out, mesh_out)
